# Initial kernel scaffold; baseline (speedup 1.0000x reference)
import jax
import jax.numpy as jnp
from jax.experimental import pallas as pl


def kernel(features, times, segment_ids, batch_ids, t_start, t_stop, decay_rate):
    raise NotImplementedError("write your pallas kernel here")


# speedup vs baseline: 1.2445x; 1.2445x over previous
import functools

import jax
import jax.numpy as jnp
from jax import lax
from jax.experimental import pallas as pl
from jax.experimental.pallas import tpu as pltpu
from jax.experimental.pallas import tpu_sc as plsc

F = 16
G = 4096
B = 8
E = 524288
C = 32
ROWS = B * F * G


_EB = 4096
_NBLK = E // _EB


def _tc1_body(ts_ref, dt_ref, lam_ref, t_ref, bid_ref, sid_ref, feat_ref,
              contrib_ref, ridx_ref):
    t = t_ref[0, 0, :]
    bid = bid_ref[0, 0, :]
    sid = sid_ref[0, 0, :]
    ts = jnp.zeros((_EB,), jnp.float32)
    dt = jnp.zeros((_EB,), jnp.float32)
    for b in range(B):
        ts = jnp.where(bid == b, ts_ref[b], ts)
        dt = jnp.where(bid == b, dt_ref[b], dt)
    rel = (t - ts) / dt
    frame_f = jnp.clip(jnp.ceil(rel) - 1.0, 0.0, F - 1.0)
    tau = ts + (frame_f + 1.0) * dt
    d = tau - t
    lam = lam_ref[0, :]
    w = jnp.exp(-d[:, None] * lam[None, :])
    cw = feat_ref[...] * w
    for k in range(4):
        contrib_ref[:, pl.ds(32 * k, 32)] = cw[1024 * k:1024 * (k + 1), :]
    frame = frame_f.astype(jnp.int32)
    ridx_ref[0, 0, :] = (bid * F + frame) * G + sid


def _tc1(times3, bids3, sids3, features, t_start, dt_frame, lam2):
    return pl.pallas_call(
        _tc1_body,
        grid=(_NBLK,),
        in_specs=[
            pl.BlockSpec(memory_space=pltpu.SMEM),
            pl.BlockSpec(memory_space=pltpu.SMEM),
            pl.BlockSpec((1, C), lambda i: (0, 0)),
            pl.BlockSpec((1, 1, _EB), lambda i: (i, 0, 0)),
            pl.BlockSpec((1, 1, _EB), lambda i: (i, 0, 0)),
            pl.BlockSpec((1, 1, _EB), lambda i: (i, 0, 0)),
            pl.BlockSpec((_EB, C), lambda i: (i, 0)),
        ],
        out_specs=[
            pl.BlockSpec((_EB // 4, 128), lambda i: (i, 0)),
            pl.BlockSpec((1, 1, _EB), lambda i: (i, 0, 0)),
        ],
        out_shape=[
            jax.ShapeDtypeStruct((E // 4, 128), jnp.float32),
            jax.ShapeDtypeStruct((_NBLK, 1, _EB), jnp.int32),
        ],
    )(t_start, dt_frame, lam2, times3, bids3, sids3, features)


_NC = 2
_NS = 16
_PASS_ROWS = 32768
_NPASS = ROWS // _PASS_ROWS
_NPC = _NPASS // _NC
_PROWS = _PASS_ROWS // 4
_SROWS = _PROWS + 8
_ESL = E // _NS
_SEG = 8192
_TRASH = _SEG + 128
_NSEG = _ESL // _SEG
_DR = _PROWS // _NS


def _sc_body(contrib_hbm, ridx_hbm, acc_hbm, spacc, rchunk, ids_seg, rows_v,
             staged, locb, pids, sem):
    cid = lax.axis_index("c")
    sid = lax.axis_index("s")
    slice_base = sid * _ESL
    iota = lax.broadcasted_iota(jnp.int32, (16,), 0)

    def _zinit(i, _):
        z = jnp.zeros((16,), jnp.float32)
        for k in range(8):
            staged[i, pl.ds(k * 16, 16)] = z
        return 0

    lax.fori_loop(0, 128, _zinit, 0)

    def _pass(q, _):
        base_row = (cid * _NPC + q) * _PASS_ROWS
        pbase = (cid * _NPC + q) * _PROWS
        for k in range(_DR // 128):
            pltpu.sync_copy(staged, spacc.at[pl.ds(sid * _DR + k * 128, 128)])
        plsc.subcore_barrier()

        def _segment(seg, _2):
            seg_base = slice_base + seg * _SEG
            pltpu.sync_copy(ridx_hbm.at[pl.ds(seg_base, _SEG)], rchunk)

            def _scan(i, carry, base_row=base_row):
                cnt, gid = carry
                r = rchunk[pl.ds(i * 16, 16)]
                loc = r - base_row
                s_lo = plsc.bitcast(loc, jnp.uint32) >> 31
                s_hi = plsc.bitcast(loc - _PASS_ROWS, jnp.uint32) >> 31
                oki = (s_hi * (1 - s_lo)).astype(jnp.int32)
                pos = cnt + plsc.cumsum(oki) - 1
                posf = pos * oki + _TRASH * (1 - oki)
                plsc.store_scatter(ids_seg, [posf], gid)
                return cnt + jnp.sum(oki), gid + 16

            cnt, _ = lax.fori_loop(0, _SEG // 16, _scan,
                                   (jnp.int32(0), seg_base + iota),
                                   unroll=4)
            for j in range(8):
                plsc.store_scatter(
                    ids_seg, [cnt + j * 16 + iota],
                    jnp.full((16,), seg_base, jnp.int32))

            def _blk(blk, _, base_row=base_row, seg_base=seg_base, cnt=cnt):
                def _pids(g, c):
                    gids = ids_seg[pl.ds(blk * 128 + g * 16, 16)]
                    pids[0, pl.ds(g * 16, 16)] = ((gids >> 12) << 10) | (
                        gids & 1023)
                    return c

                lax.fori_loop(0, 8, _pids, 0)
                pltpu.async_copy(contrib_hbm.at[pids.at[0]], rows_v,
                                 sem).wait()

                def _group(g, fill):
                    gids = ids_seg[pl.ds(blk * 128 + g * 16, 16)]
                    locs = plsc.load_gather(rchunk, [gids - seg_base])
                    locs = locs - base_row
                    s_lo = plsc.bitcast(locs, jnp.uint32) >> 31
                    s_hi = plsc.bitcast(locs - _PASS_ROWS, jnp.uint32) >> 31
                    posn = blk * 128 + g * 16 + iota
                    real = plsc.bitcast(posn - cnt, jnp.uint32) >> 31
                    inb = (s_hi * (1 - s_lo) * real).astype(jnp.int32)
                    locf = locs * inb
                    lane0 = ((locf >> 13) * 32) * inb
                    src0 = ((gids >> 10) & 3) * 32
                    if fill:
                        locb[0, pl.ds(g * 16, 16)] = (
                            (locf & 8191) * inb + _PROWS * (1 - inb))
                    z = jnp.zeros((16,), jnp.float32)
                    for t in range(16):
                        e = g * 16 + t
                        l0 = lane0[t]
                        ev = jnp.full((16,), e, jnp.int32)
                        if fill:
                            s0 = src0[t]
                            v0 = plsc.load_gather(rows_v, [ev, s0 + iota])
                            v1 = plsc.load_gather(rows_v,
                                                  [ev, s0 + 16 + iota])
                            plsc.store_scatter(staged, [ev, l0 + iota], v0)
                            plsc.store_scatter(staged, [ev, l0 + 16 + iota],
                                               v1)
                        else:
                            plsc.store_scatter(staged, [ev, l0 + iota], z)
                            plsc.store_scatter(staged, [ev, l0 + 16 + iota],
                                               z)
                    return fill

                lax.fori_loop(0, 8, lambda g, c: _group(g, True) and 0, 0)
                pltpu.sync_copy(staged, spacc.at[locb.at[0]], add=True)
                lax.fori_loop(0, 8, lambda g, c: _group(g, False) or 0, 0)
                return 0

            lax.fori_loop(0, (cnt + 127) // 128, _blk, 0)
            return 0

        lax.fori_loop(0, _NSEG, _segment, 0)
        plsc.subcore_barrier()
        pltpu.sync_copy(spacc.at[pl.ds(sid * _DR, _DR)],
                        acc_hbm.at[pl.ds(pbase + sid * _DR, _DR)])
        plsc.subcore_barrier()
        return 0

    lax.fori_loop(0, _NPC, _pass, 0)


@functools.cache
def _sc_accum():
    return pl.kernel(
        _sc_body,
        out_type=jax.ShapeDtypeStruct((ROWS // 4, 128), jnp.float32),
        mesh=plsc.VectorSubcoreMesh(core_axis_name="c",
                                    subcore_axis_name="s"),
        compiler_params=pltpu.CompilerParams(needs_layout_passes=False),
        scratch_types=[
            pltpu.VMEM_SHARED((_SROWS, 128), jnp.float32),
            pltpu.VMEM((_SEG,), jnp.int32),
            pltpu.VMEM((_SEG + 144,), jnp.int32),
            pltpu.VMEM((128, 128), jnp.float32),
            pltpu.VMEM((128, 128), jnp.float32),
            pltpu.VMEM((1, 128), jnp.int32),
            pltpu.VMEM((1, 128), jnp.int32),
            pltpu.SemaphoreType.DMA,
        ],
    )


_GB = 512
_NGB = G // _GB


def _tc2_body(dec_ref, *refs):
    acc_refs = refs[:4]
    y_ref = refs[4]
    dec = dec_ref[0, 0, 0, :]
    xs = [acc_refs[i][...] for i in range(4)]

    def frame_slice(f):
        grp = (f // 8) * 2 + (f % 2)
        r = (f % 8) // 2
        return xs[grp][:, 32 * r:32 * (r + 1)]

    cur = frame_slice(0)
    y_ref[0, 0] = cur
    for f in range(1, F):
        cur = cur * dec[None, :] + frame_slice(f)
        y_ref[0, f] = cur


def _tc2(dec4, accp):
    nrb = _GB

    def _mk_map(a, s):
        def _map(b, g, a=a, s=s):
            row_blk = ((2 * b + a) * 8192 + s * 4096) // nrb + g
            return (row_blk, 0)
        return _map

    in_specs = [pl.BlockSpec((1, 1, 1, C), lambda b, g: (b, 0, 0, 0))]
    for a in range(2):
        for s in range(2):
            in_specs.append(pl.BlockSpec((nrb, 128), _mk_map(a, s)))
    return pl.pallas_call(
        _tc2_body,
        grid=(B, _NGB),
        in_specs=in_specs,
        out_specs=pl.BlockSpec((1, F, _GB, C), lambda b, g: (b, 0, g, 0)),
        out_shape=jax.ShapeDtypeStruct((B, F, G, C), jnp.float32),
    )(dec4, *([accp] * 4))


def kernel(features, times, segment_ids, batch_ids, t_start, t_stop,
           decay_rate):
    lam = jax.nn.softplus(decay_rate)
    dt_frame = (t_stop - t_start) / F
    dec4 = jnp.exp(-dt_frame[:, None] * lam[None, :]).reshape(B, 1, 1, C)
    times3 = times.reshape(_NBLK, 1, _EB)
    bids3 = batch_ids.reshape(_NBLK, 1, _EB)
    sids3 = segment_ids.reshape(_NBLK, 1, _EB)
    contrib, ridx3 = _tc1(times3, bids3, sids3, features, t_start, dt_frame,
                          lam.reshape(1, C))
    accp = _sc_accum()(contrib, ridx3.reshape(E))
    return _tc2(dec4, accp)
